# baseline (speedup 1.0000x reference)
import jax
import jax.numpy as jnp
from jax.experimental import pallas as pl
from jax.experimental.pallas import tpu as pltpu

EXPERT_INTER = 1408
HIDDEN = 2048
TOP_K = 8

HB1 = EXPERT_INTER
HB2 = HIDDEN // 2


def _fused_kernel(idx_ref, w_ref, x_ref, gate_ref, up_ref, down_ref, o_ref,
                  inter_s, out_s):
    k = pl.program_id(0)
    s = pl.program_id(1)

    @pl.when(jnp.logical_and(k == 0, s == 0))
    def _zero():
        out_s[...] = jnp.zeros_like(out_s)

    @pl.when(s == 0)
    def _phase1():
        g = jax.lax.dot_general(
            x_ref[...], gate_ref[0],
            (((1,), (1,)), ((), ())),
            preferred_element_type=jnp.float32,
        )
        u = jax.lax.dot_general(
            x_ref[...], up_ref[0],
            (((1,), (1,)), ((), ())),
            preferred_element_type=jnp.float32,
        )
        val = (g * jax.nn.sigmoid(g)) * u * w_ref[k]

        inter_s[...] = val

    @pl.when(s >= 1)
    def _phase2():
        part = jax.lax.dot_general(
            inter_s[...], down_ref[0],
            (((1,), (1,)), ((), ())),
            preferred_element_type=jnp.float32,
        )

        @pl.when(s == 1)
        def _lo():
            out_s[:, 0:HB2] += part

        @pl.when(s == 2)
        def _hi():
            out_s[:, HB2:HIDDEN] += part

    @pl.when(jnp.logical_and(k == TOP_K - 1, s == 2))
    def _emit():
        o_ref[...] = out_s[...]


def kernel(x_bc1t, topk_idx, topk_weights, gate_up_all, down_all):
    x = x_bc1t.reshape(1, HIDDEN)
    idx = topk_idx.astype(jnp.int32)

    out = pl.pallas_call(
        _fused_kernel,
        grid_spec=pltpu.PrefetchScalarGridSpec(
            num_scalar_prefetch=2,
            grid=(TOP_K, 3),
            in_specs=[
                pl.BlockSpec((1, HIDDEN), lambda k, s, idx, w: (0, 0)),
                pl.BlockSpec(
                    (1, HB1, HIDDEN),
                    lambda k, s, idx, w: (idx[k], 0, 0)),
                pl.BlockSpec(
                    (1, HB1, HIDDEN),
                    lambda k, s, idx, w: (idx[k], 1, 0)),
                pl.BlockSpec(
                    (1, HB2, EXPERT_INTER),
                    lambda k, s, idx, w: (idx[k], jnp.maximum(s - 1, 0), 0)),
            ],
            out_specs=pl.BlockSpec((1, HIDDEN), lambda k, s, idx, w: (0, 0)),
            scratch_shapes=[
                pltpu.VMEM((1, EXPERT_INTER), jnp.float32),
                pltpu.VMEM((1, HIDDEN), jnp.float32),
            ],
        ),
        out_shape=jax.ShapeDtypeStruct((1, HIDDEN), jnp.float32),
    )(idx, topk_weights, x, gate_up_all, gate_up_all, down_all)

    return out.reshape(1, HIDDEN, 1, 1)


# speedup vs baseline: 1.2782x; 1.2782x over previous
import jax
import jax.numpy as jnp
from jax.experimental import pallas as pl
from jax.experimental.pallas import tpu as pltpu

EXPERT_INTER = 1408
HIDDEN = 2048
TOP_K = 8


def _inter_kernel(idx_ref, w_ref, x_ref, gu_ref, o_ref):
    k = pl.program_id(0)
    gu = jax.lax.dot_general(
        x_ref[...], gu_ref[0],
        (((1,), (1,)), ((), ())),
        preferred_element_type=jnp.float32,
    )
    g = gu[:, :EXPERT_INTER]
    u = gu[:, EXPERT_INTER:]
    o_ref[0] = (g * jax.nn.sigmoid(g)) * u * w_ref[k]


def _down_kernel(idx_ref, w_ref, inter_ref, down_ref, o_ref):
    k = pl.program_id(1)
    part = jax.lax.dot_general(
        inter_ref[0], down_ref[0],
        (((1,), (1,)), ((), ())),
        preferred_element_type=jnp.float32,
    )

    @pl.when(k == 0)
    def _init():
        o_ref[...] = part

    @pl.when(k > 0)
    def _acc():
        o_ref[...] += part


def kernel(x_bc1t, topk_idx, topk_weights, gate_up_all, down_all):
    x = x_bc1t.reshape(1, HIDDEN)
    idx = topk_idx.astype(jnp.int32)

    inter = pl.pallas_call(
        _inter_kernel,
        grid_spec=pltpu.PrefetchScalarGridSpec(
            num_scalar_prefetch=2,
            grid=(TOP_K,),
            in_specs=[
                pl.BlockSpec((1, HIDDEN), lambda k, idx, w: (0, 0)),
                pl.BlockSpec((1, 2 * EXPERT_INTER, HIDDEN),
                             lambda k, idx, w: (idx[k], 0, 0)),
            ],
            out_specs=pl.BlockSpec((1, 1, EXPERT_INTER), lambda k, idx, w: (k, 0, 0)),
        ),
        out_shape=jax.ShapeDtypeStruct((TOP_K, 1, EXPERT_INTER), jnp.float32),
    )(idx, topk_weights, x, gate_up_all)

    out = pl.pallas_call(
        _down_kernel,
        grid_spec=pltpu.PrefetchScalarGridSpec(
            num_scalar_prefetch=2,
            grid=(1, TOP_K),
            in_specs=[
                pl.BlockSpec((1, 1, EXPERT_INTER), lambda b, k, idx, w: (k, 0, 0)),
                pl.BlockSpec((1, HIDDEN, EXPERT_INTER),
                             lambda b, k, idx, w: (idx[k], b, 0)),
            ],
            out_specs=pl.BlockSpec((1, HIDDEN), lambda b, k, idx, w: (0, b)),
        ),
        out_shape=jax.ShapeDtypeStruct((1, HIDDEN), jnp.float32),
    )(idx, topk_weights, inter, down_all)

    return out.reshape(1, HIDDEN, 1, 1)


# speedup vs baseline: 1.2860x; 1.0061x over previous
import jax
import jax.numpy as jnp
from jax.experimental import pallas as pl
from jax.experimental.pallas import tpu as pltpu

EXPERT_INTER = 1408
HIDDEN = 2048
TOP_K = 8

RB1 = 1408
RB2 = 2048


def _inter_kernel(idx_ref, w_ref, x_ref, gate_ref, up_ref, o_ref):
    k = pl.program_id(0)
    g = jax.lax.dot_general(
        x_ref[...], gate_ref[0],
        (((1,), (1,)), ((), ())),
        preferred_element_type=jnp.float32,
    )
    u = jax.lax.dot_general(
        x_ref[...], up_ref[0],
        (((1,), (1,)), ((), ())),
        preferred_element_type=jnp.float32,
    )
    o_ref[0] = (g * jax.nn.sigmoid(g)) * u * w_ref[k]


def _down_kernel(idx_ref, w_ref, inter_ref, down_ref, o_ref):
    k = pl.program_id(1)
    part = jax.lax.dot_general(
        inter_ref[0], down_ref[0],
        (((1,), (1,)), ((), ())),
        preferred_element_type=jnp.float32,
    )

    @pl.when(k == 0)
    def _init():
        o_ref[...] = part

    @pl.when(k > 0)
    def _acc():
        o_ref[...] += part


def kernel(x_bc1t, topk_idx, topk_weights, gate_up_all, down_all):
    x = x_bc1t.reshape(1, HIDDEN)
    idx = topk_idx.astype(jnp.int32)
    nb1 = EXPERT_INTER // RB1
    nb2 = HIDDEN // RB2

    inter = pl.pallas_call(
        _inter_kernel,
        grid_spec=pltpu.PrefetchScalarGridSpec(
            num_scalar_prefetch=2,
            grid=(TOP_K, nb1),
            in_specs=[
                pl.BlockSpec((1, HIDDEN), lambda k, b, idx, w: (0, 0)),
                pl.BlockSpec((1, RB1, HIDDEN),
                             lambda k, b, idx, w: (idx[k], b, 0)),
                pl.BlockSpec((1, RB1, HIDDEN),
                             lambda k, b, idx, w: (idx[k], b + EXPERT_INTER // RB1, 0)),
            ],
            out_specs=pl.BlockSpec((1, 1, RB1), lambda k, b, idx, w: (k, 0, b)),
        ),
        out_shape=jax.ShapeDtypeStruct((TOP_K, 1, EXPERT_INTER), jnp.float32),
    )(idx, topk_weights, x, gate_up_all, gate_up_all)

    out = pl.pallas_call(
        _down_kernel,
        grid_spec=pltpu.PrefetchScalarGridSpec(
            num_scalar_prefetch=2,
            grid=(nb2, TOP_K),
            in_specs=[
                pl.BlockSpec((1, 1, EXPERT_INTER), lambda b, k, idx, w: (k, 0, 0)),
                pl.BlockSpec((1, RB2, EXPERT_INTER),
                             lambda b, k, idx, w: (idx[k], b, 0)),
            ],
            out_specs=pl.BlockSpec((1, RB2), lambda b, k, idx, w: (0, b)),
        ),
        out_shape=jax.ShapeDtypeStruct((1, HIDDEN), jnp.float32),
    )(idx, topk_weights, inter, down_all)

    return out.reshape(1, HIDDEN, 1, 1)
